# Initial kernel scaffold; baseline (speedup 1.0000x reference)
import jax
import jax.numpy as jnp
from jax.experimental import pallas as pl


def kernel(x, edge_index, W0, b0, g0, be0, W1, b1, g1, be1, W2, b2, g2, be2, Wc1, bc1, Wc2, bc2):
    raise NotImplementedError("write your pallas kernel here")


# speedup vs baseline: 11.2198x; 11.2198x over previous
import functools
import math

import jax
import jax.numpy as jnp
from jax import lax
from jax.experimental import pallas as pl
from jax.experimental.pallas import tpu as pltpu
from jax.experimental.pallas import tpu_sc as plsc

N = 10000
E = 320000
D = 128
H = 128
C = 4
EPS = 1e-5

NC = 2
NS = 16
NW = NC * NS
EPW = E // NW
CHUNK = 80
NCHUNK = EPW // CHUNK
NPAD = 10240
DPW = NPAD // NS
ROWS_PER_SUB = NPAD // NS
STAGE_ROWS = 128

_sc_mesh = plsc.VectorSubcoreMesh(core_axis_name="c", subcore_axis_name="s")


@functools.partial(
    pl.kernel,
    out_type=jax.ShapeDtypeStruct((NC * NPAD,), jnp.float32),
    mesh=_sc_mesh,
    scratch_types=[
        pltpu.VMEM((CHUNK,), jnp.int32),
        pltpu.VMEM((CHUNK,), jnp.float32),
        pltpu.VMEM((DPW,), jnp.float32),
        pltpu.VMEM_SHARED((NPAD,), jnp.float32),
    ],
)
def _deg(dst_hbm, out_hbm, didx, ones, stage, deg_sh):
    core = lax.axis_index("c")
    sub = lax.axis_index("s")
    wid = core * NS + sub

    def _init(i, _):
        ones[pl.ds(i * 16, 16)] = jnp.ones((16,), jnp.float32)
        return 0

    lax.fori_loop(0, CHUNK // 16, _init, 0)

    def _zstage(i, _):
        stage[pl.ds(i * 16, 16)] = jnp.zeros((16,), jnp.float32)
        return 0

    lax.fori_loop(0, DPW // 16, _zstage, 0)
    sl = pl.ds(sub * DPW, DPW)
    pltpu.sync_copy(stage, deg_sh.at[sl])
    plsc.subcore_barrier()

    base = wid * EPW

    def _body(j, _):
        pltpu.sync_copy(dst_hbm.at[pl.ds(base + j * CHUNK, CHUNK)], didx)
        pltpu.sync_copy(ones, deg_sh.at[didx], add=True)
        return 0

    lax.fori_loop(0, NCHUNK, _body, 0)
    plsc.subcore_barrier()
    pltpu.sync_copy(deg_sh.at[sl], stage)
    pltpu.sync_copy(stage, out_hbm.at[pl.ds(core * NPAD + sub * DPW, DPW)])


@functools.partial(
    pl.kernel,
    out_type=jax.ShapeDtypeStruct((NC, NPAD, D), jnp.float32),
    mesh=_sc_mesh,
    scratch_types=[
        pltpu.VMEM((CHUNK,), jnp.int32),
        pltpu.VMEM((CHUNK,), jnp.int32),
        pltpu.VMEM((CHUNK, D), jnp.float32),
        pltpu.VMEM((STAGE_ROWS, D), jnp.float32),
        pltpu.VMEM_SHARED((NPAD, D), jnp.float32),
        pltpu.SemaphoreType.DMA,
    ],
)
def _agg(y_hbm, src_hbm, dst_hbm, out_hbm, sidx, didx, rows, stage, acc_sh, sem):
    core = lax.axis_index("c")
    sub = lax.axis_index("s")
    wid = core * NS + sub

    def _z(i, _):
        r = i // (D // 16)
        cc = i % (D // 16)
        stage[r, pl.ds(cc * 16, 16)] = jnp.zeros((16,), jnp.float32)
        return 0

    lax.fori_loop(0, STAGE_ROWS * (D // 16), _z, 0)

    def _zc(t, _):
        pltpu.sync_copy(
            stage, acc_sh.at[pl.ds(sub * ROWS_PER_SUB + t * STAGE_ROWS, STAGE_ROWS)])
        return 0

    lax.fori_loop(0, ROWS_PER_SUB // STAGE_ROWS, _zc, 0)
    plsc.subcore_barrier()

    base = wid * EPW

    def _body(j, _):
        off = base + j * CHUNK
        pltpu.sync_copy(src_hbm.at[pl.ds(off, CHUNK)], sidx)
        pltpu.sync_copy(dst_hbm.at[pl.ds(off, CHUNK)], didx)
        pltpu.async_copy(y_hbm.at[sidx], rows, sem).wait()
        pltpu.sync_copy(rows, acc_sh.at[didx], add=True)
        return 0

    lax.fori_loop(0, NCHUNK, _body, 0)
    plsc.subcore_barrier()

    def _wb(t, _):
        sl = pl.ds(sub * ROWS_PER_SUB + t * STAGE_ROWS, STAGE_ROWS)
        pltpu.sync_copy(acc_sh.at[sl], stage)
        pltpu.sync_copy(stage, out_hbm.at[core, sl])
        return 0

    lax.fori_loop(0, ROWS_PER_SUB // STAGE_ROWS, _wb, 0)


BR = 1000
GRID = N // BR
_BNS = 1.0 / math.sqrt(1.0 + EPS)


def _t0_body(x_ref, w_ref, d0_ref, d1_ref, y_ref, dinv_ref):
    deg = d0_ref[...] + d1_ref[...] + 1.0
    dinv = lax.rsqrt(deg)
    dinv_ref[...] = dinv
    y_ref[...] = jnp.dot(x_ref[...], w_ref[...],
                         preferred_element_type=jnp.float32) * dinv


_t0 = pl.pallas_call(
    _t0_body,
    grid=(GRID,),
    in_specs=[
        pl.BlockSpec((BR, D), lambda i: (i, 0)),
        pl.BlockSpec((D, H), lambda i: (0, 0)),
        pl.BlockSpec((BR, 1), lambda i: (i, 0)),
        pl.BlockSpec((BR, 1), lambda i: (i, 0)),
    ],
    out_specs=[
        pl.BlockSpec((BR, H), lambda i: (i, 0)),
        pl.BlockSpec((BR, 1), lambda i: (i, 0)),
    ],
    out_shape=[
        jax.ShapeDtypeStruct((N, H), jnp.float32),
        jax.ShapeDtypeStruct((N, 1), jnp.float32),
    ],
)


def _tmid_body(acc_ref, y_ref, dinv_ref, b_ref, g_ref, be_ref, wn_ref, out_ref):
    dinv = dinv_ref[...]
    s = acc_ref[0] + acc_ref[1] + y_ref[...]
    z = s * dinv + b_ref[...]
    h = jnp.maximum(z * (g_ref[...] * _BNS) + be_ref[...], 0.0)
    out_ref[...] = jnp.dot(h, wn_ref[...],
                           preferred_element_type=jnp.float32) * dinv


_tmid = pl.pallas_call(
    _tmid_body,
    grid=(GRID,),
    in_specs=[
        pl.BlockSpec((NC, BR, H), lambda i: (0, i, 0)),
        pl.BlockSpec((BR, H), lambda i: (i, 0)),
        pl.BlockSpec((BR, 1), lambda i: (i, 0)),
        pl.BlockSpec((1, H), lambda i: (0, 0)),
        pl.BlockSpec((1, H), lambda i: (0, 0)),
        pl.BlockSpec((1, H), lambda i: (0, 0)),
        pl.BlockSpec((H, H), lambda i: (0, 0)),
    ],
    out_specs=pl.BlockSpec((BR, H), lambda i: (i, 0)),
    out_shape=jax.ShapeDtypeStruct((N, H), jnp.float32),
)


def _t3_body(acc_ref, y_ref, dinv_ref, b_ref, g_ref, be_ref,
             wc1_ref, bc1_ref, wc2_ref, bc2_ref, out_ref, ms_ref):
    i = pl.program_id(0)
    s = acc_ref[0] + acc_ref[1] + y_ref[...]
    z = s * dinv_ref[...] + b_ref[...]
    h = jnp.maximum(z * (g_ref[...] * _BNS) + be_ref[...], 0.0)
    ps = jnp.sum(h, axis=0, keepdims=True)

    @pl.when(i == 0)
    def _():
        ms_ref[...] = ps

    @pl.when(i > 0)
    def _():
        ms_ref[...] += ps

    @pl.when(i == GRID - 1)
    def _():
        m = ms_ref[...] * (1.0 / N)
        t = jnp.maximum(
            jnp.dot(m, wc1_ref[...], preferred_element_type=jnp.float32)
            + bc1_ref[...], 0.0)
        out_ref[...] = (jnp.dot(t, wc2_ref[...],
                                preferred_element_type=jnp.float32)
                        + bc2_ref[...])


_t3 = pl.pallas_call(
    _t3_body,
    grid=(GRID,),
    in_specs=[
        pl.BlockSpec((NC, BR, H), lambda i: (0, i, 0)),
        pl.BlockSpec((BR, H), lambda i: (i, 0)),
        pl.BlockSpec((BR, 1), lambda i: (i, 0)),
        pl.BlockSpec((1, H), lambda i: (0, 0)),
        pl.BlockSpec((1, H), lambda i: (0, 0)),
        pl.BlockSpec((1, H), lambda i: (0, 0)),
        pl.BlockSpec((H, H // 2), lambda i: (0, 0)),
        pl.BlockSpec((1, H // 2), lambda i: (0, 0)),
        pl.BlockSpec((H // 2, C), lambda i: (0, 0)),
        pl.BlockSpec((1, C), lambda i: (0, 0)),
    ],
    out_specs=pl.BlockSpec((1, C), lambda i: (0, 0)),
    out_shape=jax.ShapeDtypeStruct((1, C), jnp.float32),
    scratch_shapes=[pltpu.VMEM((1, H), jnp.float32)],
)


def kernel(x, edge_index, W0, b0, g0, be0, W1, b1, g1, be1,
           W2, b2, g2, be2, Wc1, bc1, Wc2, bc2):
    src = edge_index[0]
    dst = edge_index[1]

    degp = _deg(dst)
    d0 = degp[:N].reshape(N, 1)
    d1 = degp[NPAD:NPAD + N].reshape(N, 1)

    y0, dinv = _t0(x, W0, d0, d1)
    a0 = _agg(y0, src, dst)
    y1 = _tmid(a0, y0, dinv, b0.reshape(1, H), g0.reshape(1, H),
               be0.reshape(1, H), W1)
    a1 = _agg(y1, src, dst)
    y2 = _tmid(a1, y1, dinv, b1.reshape(1, H), g1.reshape(1, H),
               be1.reshape(1, H), W2)
    a2 = _agg(y2, src, dst)
    out = _t3(a2, y2, dinv, b2.reshape(1, H), g2.reshape(1, H),
              be2.reshape(1, H), Wc1, bc1.reshape(1, H // 2),
              Wc2, bc2.reshape(1, C))
    return out


# speedup vs baseline: 21.7859x; 1.9417x over previous
import functools
import math

import jax
import jax.numpy as jnp
from jax import lax
from jax.experimental import pallas as pl
from jax.experimental.pallas import tpu as pltpu
from jax.experimental.pallas import tpu_sc as plsc

N = 10000
E = 320000
D = 128
H = 128
C = 4
EPS = 1e-5

NC = 2
NS = 16
NW = NC * NS
EPW = E // NW
CHUNK = 80
NCHUNK = EPW // CHUNK
NPAD = 10240
DPW = NPAD // NS
ROWS_PER_SUB = NPAD // NS
DEPTH = 4
PRE = 2

_sc_mesh = plsc.VectorSubcoreMesh(core_axis_name="c", subcore_axis_name="s")


@functools.partial(
    pl.kernel,
    out_type=jax.ShapeDtypeStruct((NC * NPAD,), jnp.float32),
    mesh=_sc_mesh,
    scratch_types=[
        pltpu.VMEM((NCHUNK, CHUNK), jnp.int32),
        pltpu.VMEM((CHUNK,), jnp.float32),
        pltpu.VMEM((DPW,), jnp.float32),
        pltpu.VMEM_SHARED((NPAD,), jnp.float32),
        pltpu.SemaphoreType.DMA,
    ],
)
def _deg(dst_hbm, out_hbm, didx_all, ones, stage, deg_sh, ssem):
    core = lax.axis_index("c")
    sub = lax.axis_index("s")
    wid = core * NS + sub

    def _init(i, _):
        ones[pl.ds(i * 16, 16)] = jnp.ones((16,), jnp.float32)
        return 0

    lax.fori_loop(0, CHUNK // 16, _init, 0)

    def _zstage(i, _):
        stage[pl.ds(i * 16, 16)] = jnp.zeros((16,), jnp.float32)
        return 0

    lax.fori_loop(0, DPW // 16, _zstage, 0)
    sl = pl.ds(sub * DPW, DPW)
    pltpu.sync_copy(stage, deg_sh.at[sl])
    pltpu.sync_copy(dst_hbm.at[wid], didx_all)
    plsc.subcore_barrier()

    def _body(j, _):
        pltpu.async_copy(ones, deg_sh.at[didx_all.at[j]], ssem, add=True)
        return 0

    lax.fori_loop(0, NCHUNK, _body, 0)

    def _drain(j, _):
        pltpu.make_async_copy(ones, deg_sh.at[didx_all.at[j]], ssem).wait()
        return 0

    lax.fori_loop(0, NCHUNK, _drain, 0)
    plsc.subcore_barrier()
    pltpu.sync_copy(deg_sh.at[sl], stage)
    pltpu.sync_copy(stage, out_hbm.at[pl.ds(core * NPAD + sub * DPW, DPW)])


@functools.partial(
    pl.kernel,
    out_type=jax.ShapeDtypeStruct((NC, NPAD, D), jnp.float32),
    mesh=_sc_mesh,
    scratch_types=[
        pltpu.VMEM((DEPTH, CHUNK), jnp.int32),
        pltpu.VMEM((DEPTH, CHUNK), jnp.int32),
        pltpu.VMEM((DEPTH, CHUNK, D), jnp.float32),
        pltpu.VMEM_SHARED((NPAD, D), jnp.float32),
        pltpu.SemaphoreType.DMA((DEPTH,)),
        pltpu.SemaphoreType.DMA((DEPTH,)),
    ],
)
def _agg(y_hbm, src_hbm, dst_hbm, z_hbm, out_hbm, sidx_r, didx_r, rows,
         acc_sh, gsem, ssem):
    core = lax.axis_index("c")
    sub = lax.axis_index("s")
    wid = core * NS + sub

    pltpu.sync_copy(z_hbm, acc_sh.at[pl.ds(sub * ROWS_PER_SUB, ROWS_PER_SUB)])
    plsc.subcore_barrier()

    for b in range(PRE):
        pltpu.sync_copy(src_hbm.at[wid, b], sidx_r.at[b])
        pltpu.sync_copy(dst_hbm.at[wid, b], didx_r.at[b])
        pltpu.async_copy(y_hbm.at[sidx_r.at[b]], rows.at[b], gsem.at[b])

    def _body(j, _):
        s = lax.rem(j, DEPTH)
        pltpu.make_async_copy(y_hbm.at[sidx_r.at[s]], rows.at[s],
                              gsem.at[s]).wait()
        pltpu.async_copy(rows.at[s], acc_sh.at[didx_r.at[s]], ssem.at[s],
                         add=True)
        jj = j + PRE
        sp = lax.rem(jj, DEPTH)

        @pl.when(jj < NCHUNK)
        def _():
            @pl.when(jj >= DEPTH)
            def _():
                pltpu.make_async_copy(rows.at[sp], acc_sh.at[didx_r.at[sp]],
                                      ssem.at[sp]).wait()

            pltpu.sync_copy(src_hbm.at[wid, jj], sidx_r.at[sp])
            pltpu.sync_copy(dst_hbm.at[wid, jj], didx_r.at[sp])
            pltpu.async_copy(y_hbm.at[sidx_r.at[sp]], rows.at[sp],
                             gsem.at[sp])

        return 0

    lax.fori_loop(0, NCHUNK, _body, 0)

    def _drain(t, _):
        s = lax.rem(NCHUNK - DEPTH + t, DEPTH)
        pltpu.make_async_copy(rows.at[s], acc_sh.at[didx_r.at[s]],
                              ssem.at[s]).wait()
        return 0

    lax.fori_loop(0, DEPTH, _drain, 0)
    plsc.subcore_barrier()

    sl = pl.ds(sub * ROWS_PER_SUB, ROWS_PER_SUB)
    pltpu.sync_copy(acc_sh.at[sl], out_hbm.at[core, sl])


BR = 1000
GRID = N // BR
_BNS = 1.0 / math.sqrt(1.0 + EPS)


def _t0_body(x_ref, w_ref, d0_ref, d1_ref, y_ref, dinv_ref):
    deg = d0_ref[...] + d1_ref[...] + 1.0
    dinv = lax.rsqrt(deg)
    dinv_ref[...] = dinv
    y_ref[...] = jnp.dot(x_ref[...], w_ref[...],
                         preferred_element_type=jnp.float32) * dinv


_t0 = pl.pallas_call(
    _t0_body,
    grid=(GRID,),
    in_specs=[
        pl.BlockSpec((BR, D), lambda i: (i, 0)),
        pl.BlockSpec((D, H), lambda i: (0, 0)),
        pl.BlockSpec((BR, 1), lambda i: (i, 0)),
        pl.BlockSpec((BR, 1), lambda i: (i, 0)),
    ],
    out_specs=[
        pl.BlockSpec((BR, H), lambda i: (i, 0)),
        pl.BlockSpec((BR, 1), lambda i: (i, 0)),
    ],
    out_shape=[
        jax.ShapeDtypeStruct((N, H), jnp.float32),
        jax.ShapeDtypeStruct((N, 1), jnp.float32),
    ],
)


def _tmid_body(acc_ref, y_ref, dinv_ref, b_ref, g_ref, be_ref, wn_ref, out_ref):
    dinv = dinv_ref[...]
    s = acc_ref[0] + acc_ref[1] + y_ref[...]
    z = s * dinv + b_ref[...]
    h = jnp.maximum(z * (g_ref[...] * _BNS) + be_ref[...], 0.0)
    out_ref[...] = jnp.dot(h, wn_ref[...],
                           preferred_element_type=jnp.float32) * dinv


_tmid = pl.pallas_call(
    _tmid_body,
    grid=(GRID,),
    in_specs=[
        pl.BlockSpec((NC, BR, H), lambda i: (0, i, 0)),
        pl.BlockSpec((BR, H), lambda i: (i, 0)),
        pl.BlockSpec((BR, 1), lambda i: (i, 0)),
        pl.BlockSpec((1, H), lambda i: (0, 0)),
        pl.BlockSpec((1, H), lambda i: (0, 0)),
        pl.BlockSpec((1, H), lambda i: (0, 0)),
        pl.BlockSpec((H, H), lambda i: (0, 0)),
    ],
    out_specs=pl.BlockSpec((BR, H), lambda i: (i, 0)),
    out_shape=jax.ShapeDtypeStruct((N, H), jnp.float32),
)


def _t3_body(acc_ref, y_ref, dinv_ref, b_ref, g_ref, be_ref,
             wc1_ref, bc1_ref, wc2_ref, bc2_ref, out_ref, ms_ref):
    i = pl.program_id(0)
    s = acc_ref[0] + acc_ref[1] + y_ref[...]
    z = s * dinv_ref[...] + b_ref[...]
    h = jnp.maximum(z * (g_ref[...] * _BNS) + be_ref[...], 0.0)
    ps = jnp.sum(h, axis=0, keepdims=True)

    @pl.when(i == 0)
    def _():
        ms_ref[...] = ps

    @pl.when(i > 0)
    def _():
        ms_ref[...] += ps

    @pl.when(i == GRID - 1)
    def _():
        m = ms_ref[...] * (1.0 / N)
        t = jnp.maximum(
            jnp.dot(m, wc1_ref[...], preferred_element_type=jnp.float32)
            + bc1_ref[...], 0.0)
        out_ref[...] = (jnp.dot(t, wc2_ref[...],
                                preferred_element_type=jnp.float32)
                        + bc2_ref[...])


_t3 = pl.pallas_call(
    _t3_body,
    grid=(GRID,),
    in_specs=[
        pl.BlockSpec((NC, BR, H), lambda i: (0, i, 0)),
        pl.BlockSpec((BR, H), lambda i: (i, 0)),
        pl.BlockSpec((BR, 1), lambda i: (i, 0)),
        pl.BlockSpec((1, H), lambda i: (0, 0)),
        pl.BlockSpec((1, H), lambda i: (0, 0)),
        pl.BlockSpec((1, H), lambda i: (0, 0)),
        pl.BlockSpec((H, H // 2), lambda i: (0, 0)),
        pl.BlockSpec((1, H // 2), lambda i: (0, 0)),
        pl.BlockSpec((H // 2, C), lambda i: (0, 0)),
        pl.BlockSpec((1, C), lambda i: (0, 0)),
    ],
    out_specs=pl.BlockSpec((1, C), lambda i: (0, 0)),
    out_shape=jax.ShapeDtypeStruct((1, C), jnp.float32),
    scratch_shapes=[pltpu.VMEM((1, H), jnp.float32)],
)


def kernel(x, edge_index, W0, b0, g0, be0, W1, b1, g1, be1,
           W2, b2, g2, be2, Wc1, bc1, Wc2, bc2):
    src = edge_index[0].reshape(NW, NCHUNK, CHUNK)
    dst = edge_index[1].reshape(NW, NCHUNK, CHUNK)

    degp = _deg(dst)
    d0 = degp[:N].reshape(N, 1)
    d1 = degp[NPAD:NPAD + N].reshape(N, 1)
    zrows = jnp.zeros((ROWS_PER_SUB, D), jnp.float32)

    y0, dinv = _t0(x, W0, d0, d1)
    a0 = _agg(y0, src, dst, zrows)
    y1 = _tmid(a0, y0, dinv, b0.reshape(1, H), g0.reshape(1, H),
               be0.reshape(1, H), W1)
    a1 = _agg(y1, src, dst, zrows)
    y2 = _tmid(a1, y1, dinv, b1.reshape(1, H), g1.reshape(1, H),
               be1.reshape(1, H), W2)
    a2 = _agg(y2, src, dst, zrows)
    out = _t3(a2, y2, dinv, b2.reshape(1, H), g2.reshape(1, H),
              be2.reshape(1, H), Wc1, bc1.reshape(1, H // 2),
              Wc2, bc2.reshape(1, C))
    return out


# speedup vs baseline: 30.0060x; 1.3773x over previous
import functools
import math

import jax
import jax.numpy as jnp
from jax import lax
from jax.experimental import pallas as pl
from jax.experimental.pallas import tpu as pltpu
from jax.experimental.pallas import tpu_sc as plsc

N = 10000
E = 320000
D = 128
H = 128
C = 4
EPS = 1e-5

NC = 2
NS = 16
NW = NC * NS
EPW = E // NW
CHUNK = 80
NCHUNK = EPW // CHUNK
NPAD = 10240
DPW = NPAD // NS
ROWS_PER_SUB = NPAD // NS
DEPTH = 4
PRE = 2
IDEPTH = 8

_sc_mesh = plsc.VectorSubcoreMesh(core_axis_name="c", subcore_axis_name="s")


@functools.partial(
    pl.kernel,
    out_type=jax.ShapeDtypeStruct((NC * NPAD,), jnp.float32),
    mesh=_sc_mesh,
    scratch_types=[
        pltpu.VMEM((NCHUNK, CHUNK), jnp.int32),
        pltpu.VMEM((CHUNK,), jnp.float32),
        pltpu.VMEM((DPW,), jnp.float32),
        pltpu.VMEM_SHARED((NPAD,), jnp.float32),
        pltpu.SemaphoreType.DMA,
    ],
)
def _deg(dst_hbm, out_hbm, didx_all, ones, stage, deg_sh, ssem):
    core = lax.axis_index("c")
    sub = lax.axis_index("s")
    wid = core * NS + sub

    def _init(i, _):
        ones[pl.ds(i * 16, 16)] = jnp.ones((16,), jnp.float32)
        return 0

    lax.fori_loop(0, CHUNK // 16, _init, 0)

    def _zstage(i, _):
        stage[pl.ds(i * 16, 16)] = jnp.zeros((16,), jnp.float32)
        return 0

    lax.fori_loop(0, DPW // 16, _zstage, 0)
    sl = pl.ds(sub * DPW, DPW)
    pltpu.sync_copy(stage, deg_sh.at[sl])
    pltpu.sync_copy(dst_hbm.at[wid], didx_all)
    plsc.subcore_barrier()

    def _body(j, _):
        pltpu.async_copy(ones, deg_sh.at[didx_all.at[j]], ssem, add=True)
        return 0

    lax.fori_loop(0, NCHUNK, _body, 0)

    def _drain(j, _):
        pltpu.make_async_copy(ones, deg_sh.at[didx_all.at[j]], ssem).wait()
        return 0

    lax.fori_loop(0, NCHUNK, _drain, 0)
    plsc.subcore_barrier()
    pltpu.sync_copy(deg_sh.at[sl], stage)
    pltpu.sync_copy(stage, out_hbm.at[pl.ds(core * NPAD + sub * DPW, DPW)])


@functools.partial(
    pl.kernel,
    out_type=jax.ShapeDtypeStruct((NC, NPAD, D), jnp.float32),
    mesh=_sc_mesh,
    scratch_types=[
        pltpu.VMEM((IDEPTH, 2, CHUNK), jnp.int32),
        pltpu.VMEM((DEPTH, CHUNK, D), jnp.float32),
        pltpu.VMEM_SHARED((NPAD, D), jnp.float32),
        pltpu.SemaphoreType.DMA((IDEPTH,)),
        pltpu.SemaphoreType.DMA((DEPTH,)),
        pltpu.SemaphoreType.DMA((DEPTH,)),
    ],
)
def _agg(y_hbm, edg_hbm, z_hbm, out_hbm, exr, rows, acc_sh, isem, gsem, ssem):
    core = lax.axis_index("c")
    sub = lax.axis_index("s")
    wid = core * NS + sub

    pltpu.sync_copy(z_hbm, acc_sh.at[pl.ds(sub * ROWS_PER_SUB, ROWS_PER_SUB)])
    plsc.subcore_barrier()

    for k in range(PRE + 1):
        pltpu.sync_copy(edg_hbm.at[wid, k], exr.at[k])
    for b in range(PRE):
        pltpu.async_copy(y_hbm.at[exr.at[b, 0]], rows.at[b], gsem.at[b])

    def _body(j, _):
        s = lax.rem(j, DEPTH)
        si = lax.rem(j, IDEPTH)
        pltpu.make_async_copy(y_hbm.at[exr.at[si, 0]], rows.at[s],
                              gsem.at[s]).wait()
        pltpu.async_copy(rows.at[s], acc_sh.at[exr.at[si, 1]], ssem.at[s],
                         add=True)
        jj = j + PRE
        sp = lax.rem(jj, DEPTH)
        spi = lax.rem(jj, IDEPTH)

        @pl.when(jj < NCHUNK)
        def _():
            @pl.when(jj >= DEPTH)
            def _():
                pltpu.make_async_copy(
                    rows.at[sp], acc_sh.at[exr.at[lax.rem(jj - DEPTH, IDEPTH),
                                                  1]],
                    ssem.at[sp]).wait()

            @pl.when(jj > PRE)
            def _():
                pltpu.make_async_copy(edg_hbm.at[wid, jj], exr.at[spi],
                                      isem.at[spi]).wait()

            pltpu.async_copy(y_hbm.at[exr.at[spi, 0]], rows.at[sp],
                             gsem.at[sp])

        jn = j + PRE + 1

        @pl.when(jn < NCHUNK)
        def _():
            pltpu.async_copy(edg_hbm.at[wid, jn], exr.at[lax.rem(jn, IDEPTH)],
                             isem.at[lax.rem(jn, IDEPTH)])

        return 0

    lax.fori_loop(0, NCHUNK, _body, 0)

    def _drain(t, _):
        j = NCHUNK - DEPTH + t
        s = lax.rem(j, DEPTH)
        pltpu.make_async_copy(rows.at[s],
                              acc_sh.at[exr.at[lax.rem(j, IDEPTH), 1]],
                              ssem.at[s]).wait()
        return 0

    lax.fori_loop(0, DEPTH, _drain, 0)
    plsc.subcore_barrier()

    sl = pl.ds(sub * ROWS_PER_SUB, ROWS_PER_SUB)
    pltpu.sync_copy(acc_sh.at[sl], out_hbm.at[core, sl])


BR = 1000
GRID = N // BR
_BNS = 1.0 / math.sqrt(1.0 + EPS)


def _t0_body(x_ref, w_ref, d0_ref, d1_ref, y_ref, dinv_ref):
    deg = d0_ref[...] + d1_ref[...] + 1.0
    dinv = lax.rsqrt(deg)
    dinv_ref[...] = dinv
    y_ref[...] = jnp.dot(x_ref[...], w_ref[...],
                         preferred_element_type=jnp.float32) * dinv


_t0 = pl.pallas_call(
    _t0_body,
    grid=(GRID,),
    in_specs=[
        pl.BlockSpec((BR, D), lambda i: (i, 0)),
        pl.BlockSpec((D, H), lambda i: (0, 0)),
        pl.BlockSpec((BR, 1), lambda i: (i, 0)),
        pl.BlockSpec((BR, 1), lambda i: (i, 0)),
    ],
    out_specs=[
        pl.BlockSpec((BR, H), lambda i: (i, 0)),
        pl.BlockSpec((BR, 1), lambda i: (i, 0)),
    ],
    out_shape=[
        jax.ShapeDtypeStruct((N, H), jnp.float32),
        jax.ShapeDtypeStruct((N, 1), jnp.float32),
    ],
)


def _tmid_body(acc_ref, y_ref, dinv_ref, b_ref, g_ref, be_ref, wn_ref, out_ref):
    dinv = dinv_ref[...]
    s = acc_ref[0] + acc_ref[1] + y_ref[...]
    z = s * dinv + b_ref[...]
    h = jnp.maximum(z * (g_ref[...] * _BNS) + be_ref[...], 0.0)
    out_ref[...] = jnp.dot(h, wn_ref[...],
                           preferred_element_type=jnp.float32) * dinv


_tmid = pl.pallas_call(
    _tmid_body,
    grid=(GRID,),
    in_specs=[
        pl.BlockSpec((NC, BR, H), lambda i: (0, i, 0)),
        pl.BlockSpec((BR, H), lambda i: (i, 0)),
        pl.BlockSpec((BR, 1), lambda i: (i, 0)),
        pl.BlockSpec((1, H), lambda i: (0, 0)),
        pl.BlockSpec((1, H), lambda i: (0, 0)),
        pl.BlockSpec((1, H), lambda i: (0, 0)),
        pl.BlockSpec((H, H), lambda i: (0, 0)),
    ],
    out_specs=pl.BlockSpec((BR, H), lambda i: (i, 0)),
    out_shape=jax.ShapeDtypeStruct((N, H), jnp.float32),
)


def _t3_body(acc_ref, y_ref, dinv_ref, b_ref, g_ref, be_ref,
             wc1_ref, bc1_ref, wc2_ref, bc2_ref, out_ref, ms_ref):
    i = pl.program_id(0)
    s = acc_ref[0] + acc_ref[1] + y_ref[...]
    z = s * dinv_ref[...] + b_ref[...]
    h = jnp.maximum(z * (g_ref[...] * _BNS) + be_ref[...], 0.0)
    ps = jnp.sum(h, axis=0, keepdims=True)

    @pl.when(i == 0)
    def _():
        ms_ref[...] = ps

    @pl.when(i > 0)
    def _():
        ms_ref[...] += ps

    @pl.when(i == GRID - 1)
    def _():
        m = ms_ref[...] * (1.0 / N)
        t = jnp.maximum(
            jnp.dot(m, wc1_ref[...], preferred_element_type=jnp.float32)
            + bc1_ref[...], 0.0)
        out_ref[...] = (jnp.dot(t, wc2_ref[...],
                                preferred_element_type=jnp.float32)
                        + bc2_ref[...])


_t3 = pl.pallas_call(
    _t3_body,
    grid=(GRID,),
    in_specs=[
        pl.BlockSpec((NC, BR, H), lambda i: (0, i, 0)),
        pl.BlockSpec((BR, H), lambda i: (i, 0)),
        pl.BlockSpec((BR, 1), lambda i: (i, 0)),
        pl.BlockSpec((1, H), lambda i: (0, 0)),
        pl.BlockSpec((1, H), lambda i: (0, 0)),
        pl.BlockSpec((1, H), lambda i: (0, 0)),
        pl.BlockSpec((H, H // 2), lambda i: (0, 0)),
        pl.BlockSpec((1, H // 2), lambda i: (0, 0)),
        pl.BlockSpec((H // 2, C), lambda i: (0, 0)),
        pl.BlockSpec((1, C), lambda i: (0, 0)),
    ],
    out_specs=pl.BlockSpec((1, C), lambda i: (0, 0)),
    out_shape=jax.ShapeDtypeStruct((1, C), jnp.float32),
    scratch_shapes=[pltpu.VMEM((1, H), jnp.float32)],
)


def kernel(x, edge_index, W0, b0, g0, be0, W1, b1, g1, be1,
           W2, b2, g2, be2, Wc1, bc1, Wc2, bc2):
    edg = edge_index.reshape(2, NW, NCHUNK, CHUNK).transpose(1, 2, 0, 3)
    dst = edge_index[1].reshape(NW, NCHUNK, CHUNK)

    degp = _deg(dst)
    d0 = degp[:N].reshape(N, 1)
    d1 = degp[NPAD:NPAD + N].reshape(N, 1)
    zrows = jnp.zeros((ROWS_PER_SUB, D), jnp.float32)

    y0, dinv = _t0(x, W0, d0, d1)
    a0 = _agg(y0, edg, zrows)
    y1 = _tmid(a0, y0, dinv, b0.reshape(1, H), g0.reshape(1, H),
               be0.reshape(1, H), W1)
    a1 = _agg(y1, edg, zrows)
    y2 = _tmid(a1, y1, dinv, b1.reshape(1, H), g1.reshape(1, H),
               be1.reshape(1, H), W2)
    a2 = _agg(y2, edg, zrows)
    out = _t3(a2, y2, dinv, b2.reshape(1, H), g2.reshape(1, H),
              be2.reshape(1, H), Wc1, bc1.reshape(1, H // 2),
              Wc2, bc2.reshape(1, C))
    return out


# speedup vs baseline: 30.7479x; 1.0247x over previous
import functools
import math

import jax
import jax.numpy as jnp
from jax import lax
from jax.experimental import pallas as pl
from jax.experimental.pallas import tpu as pltpu
from jax.experimental.pallas import tpu_sc as plsc

N = 10000
E = 320000
D = 128
H = 128
C = 4
EPS = 1e-5

NC = 2
NS = 16
NW = NC * NS
EPW = E // NW
CHUNK = 80
NCHUNK = EPW // CHUNK
DCHUNK = 80
DNCHUNK = EPW // DCHUNK
NPAD = 10240
DPW = NPAD // NS
ROWS_PER_SUB = NPAD // NS
DEPTH = 4
PRE = 2
IDEPTH = 8

_sc_mesh = plsc.VectorSubcoreMesh(core_axis_name="c", subcore_axis_name="s")


@functools.partial(
    pl.kernel,
    out_type=jax.ShapeDtypeStruct((NC * NPAD,), jnp.float32),
    mesh=_sc_mesh,
    scratch_types=[
        pltpu.VMEM((DNCHUNK, DCHUNK), jnp.int32),
        pltpu.VMEM((DCHUNK,), jnp.float32),
        pltpu.VMEM((DPW,), jnp.float32),
        pltpu.VMEM_SHARED((NPAD,), jnp.float32),
        pltpu.SemaphoreType.DMA,
    ],
)
def _deg(dst_hbm, out_hbm, didx_all, ones, stage, deg_sh, ssem):
    core = lax.axis_index("c")
    sub = lax.axis_index("s")
    wid = core * NS + sub

    def _init(i, _):
        ones[pl.ds(i * 16, 16)] = jnp.ones((16,), jnp.float32)
        return 0

    lax.fori_loop(0, DCHUNK // 16, _init, 0)

    def _zstage(i, _):
        stage[pl.ds(i * 16, 16)] = jnp.zeros((16,), jnp.float32)
        return 0

    lax.fori_loop(0, DPW // 16, _zstage, 0)
    sl = pl.ds(sub * DPW, DPW)
    pltpu.sync_copy(stage, deg_sh.at[sl])
    pltpu.sync_copy(dst_hbm.at[wid], didx_all)
    plsc.subcore_barrier()

    def _body(j, _):
        pltpu.async_copy(ones, deg_sh.at[didx_all.at[j]], ssem, add=True)
        return 0

    lax.fori_loop(0, DNCHUNK, _body, 0)

    def _drain(j, _):
        pltpu.make_async_copy(ones, deg_sh.at[didx_all.at[j]], ssem).wait()
        return 0

    lax.fori_loop(0, DNCHUNK, _drain, 0)
    plsc.subcore_barrier()
    pltpu.sync_copy(deg_sh.at[sl], stage)
    pltpu.sync_copy(stage, out_hbm.at[pl.ds(core * NPAD + sub * DPW, DPW)])


@functools.partial(
    pl.kernel,
    out_type=jax.ShapeDtypeStruct((NC, NPAD, D), jnp.float32),
    mesh=_sc_mesh,
    scratch_types=[
        pltpu.VMEM((IDEPTH, 2, CHUNK), jnp.int32),
        pltpu.VMEM((DEPTH, CHUNK, D), jnp.float32),
        pltpu.VMEM_SHARED((NPAD, D), jnp.float32),
        pltpu.SemaphoreType.DMA((IDEPTH,)),
        pltpu.SemaphoreType.DMA((DEPTH,)),
        pltpu.SemaphoreType.DMA((DEPTH,)),
    ],
)
def _agg(y_hbm, edg_hbm, z_hbm, out_hbm, exr, rows, acc_sh, isem, gsem, ssem):
    core = lax.axis_index("c")
    sub = lax.axis_index("s")
    wid = core * NS + sub

    pltpu.sync_copy(z_hbm, acc_sh.at[pl.ds(sub * ROWS_PER_SUB, ROWS_PER_SUB)])

    for k in range(PRE + 1):
        pltpu.sync_copy(edg_hbm.at[wid, k], exr.at[k])
    for b in range(PRE):
        pltpu.async_copy(y_hbm.at[exr.at[b, 0]], rows.at[b], gsem.at[b])
    plsc.subcore_barrier()

    def _body(j, _):
        s = lax.rem(j, DEPTH)
        si = lax.rem(j, IDEPTH)
        pltpu.make_async_copy(y_hbm.at[exr.at[si, 0]], rows.at[s],
                              gsem.at[s]).wait()
        pltpu.async_copy(rows.at[s], acc_sh.at[exr.at[si, 1]], ssem.at[s],
                         add=True)
        jj = j + PRE
        sp = lax.rem(jj, DEPTH)
        spi = lax.rem(jj, IDEPTH)

        @pl.when(jj < NCHUNK)
        def _():
            @pl.when(jj >= DEPTH)
            def _():
                pltpu.make_async_copy(
                    rows.at[sp], acc_sh.at[exr.at[lax.rem(jj - DEPTH, IDEPTH),
                                                  1]],
                    ssem.at[sp]).wait()

            @pl.when(jj > PRE)
            def _():
                pltpu.make_async_copy(edg_hbm.at[wid, jj], exr.at[spi],
                                      isem.at[spi]).wait()

            pltpu.async_copy(y_hbm.at[exr.at[spi, 0]], rows.at[sp],
                             gsem.at[sp])

        jn = j + PRE + 1

        @pl.when(jn < NCHUNK)
        def _():
            pltpu.async_copy(edg_hbm.at[wid, jn], exr.at[lax.rem(jn, IDEPTH)],
                             isem.at[lax.rem(jn, IDEPTH)])

        return 0

    lax.fori_loop(0, NCHUNK, _body, 0)

    def _drain(t, _):
        j = NCHUNK - DEPTH + t
        s = lax.rem(j, DEPTH)
        pltpu.make_async_copy(rows.at[s],
                              acc_sh.at[exr.at[lax.rem(j, IDEPTH), 1]],
                              ssem.at[s]).wait()
        return 0

    lax.fori_loop(0, DEPTH, _drain, 0)
    plsc.subcore_barrier()

    sl = pl.ds(sub * ROWS_PER_SUB, ROWS_PER_SUB)
    pltpu.sync_copy(acc_sh.at[sl], out_hbm.at[core, sl])


BR = 2000
GRID = N // BR
_BNS = 1.0 / math.sqrt(1.0 + EPS)


def _t0_body(x_ref, w_ref, d0_ref, d1_ref, y_ref, dinv_ref):
    deg = d0_ref[...] + d1_ref[...] + 1.0
    dinv = lax.rsqrt(deg)
    dinv_ref[...] = dinv
    y_ref[...] = jnp.dot(x_ref[...], w_ref[...],
                         preferred_element_type=jnp.float32) * dinv


_t0 = pl.pallas_call(
    _t0_body,
    grid=(GRID,),
    in_specs=[
        pl.BlockSpec((BR, D), lambda i: (i, 0)),
        pl.BlockSpec((D, H), lambda i: (0, 0)),
        pl.BlockSpec((BR, 1), lambda i: (i, 0)),
        pl.BlockSpec((BR, 1), lambda i: (i, 0)),
    ],
    out_specs=[
        pl.BlockSpec((BR, H), lambda i: (i, 0)),
        pl.BlockSpec((BR, 1), lambda i: (i, 0)),
    ],
    out_shape=[
        jax.ShapeDtypeStruct((N, H), jnp.float32),
        jax.ShapeDtypeStruct((N, 1), jnp.float32),
    ],
)


def _tmid_body(acc_ref, y_ref, dinv_ref, b_ref, g_ref, be_ref, wn_ref, out_ref):
    dinv = dinv_ref[...]
    s = acc_ref[0] + acc_ref[1] + y_ref[...]
    z = s * dinv + b_ref[...]
    h = jnp.maximum(z * (g_ref[...] * _BNS) + be_ref[...], 0.0)
    out_ref[...] = jnp.dot(h, wn_ref[...],
                           preferred_element_type=jnp.float32) * dinv


_tmid = pl.pallas_call(
    _tmid_body,
    grid=(GRID,),
    in_specs=[
        pl.BlockSpec((NC, BR, H), lambda i: (0, i, 0)),
        pl.BlockSpec((BR, H), lambda i: (i, 0)),
        pl.BlockSpec((BR, 1), lambda i: (i, 0)),
        pl.BlockSpec((1, H), lambda i: (0, 0)),
        pl.BlockSpec((1, H), lambda i: (0, 0)),
        pl.BlockSpec((1, H), lambda i: (0, 0)),
        pl.BlockSpec((H, H), lambda i: (0, 0)),
    ],
    out_specs=pl.BlockSpec((BR, H), lambda i: (i, 0)),
    out_shape=jax.ShapeDtypeStruct((N, H), jnp.float32),
)


def _t3_body(acc_ref, y_ref, dinv_ref, b_ref, g_ref, be_ref,
             wc1_ref, bc1_ref, wc2_ref, bc2_ref, out_ref, ms_ref):
    i = pl.program_id(0)
    s = acc_ref[0] + acc_ref[1] + y_ref[...]
    z = s * dinv_ref[...] + b_ref[...]
    h = jnp.maximum(z * (g_ref[...] * _BNS) + be_ref[...], 0.0)
    ps = jnp.sum(h, axis=0, keepdims=True)

    @pl.when(i == 0)
    def _():
        ms_ref[...] = ps

    @pl.when(i > 0)
    def _():
        ms_ref[...] += ps

    @pl.when(i == GRID - 1)
    def _():
        m = ms_ref[...] * (1.0 / N)
        t = jnp.maximum(
            jnp.dot(m, wc1_ref[...], preferred_element_type=jnp.float32)
            + bc1_ref[...], 0.0)
        out_ref[...] = (jnp.dot(t, wc2_ref[...],
                                preferred_element_type=jnp.float32)
                        + bc2_ref[...])


_t3 = pl.pallas_call(
    _t3_body,
    grid=(GRID,),
    in_specs=[
        pl.BlockSpec((NC, BR, H), lambda i: (0, i, 0)),
        pl.BlockSpec((BR, H), lambda i: (i, 0)),
        pl.BlockSpec((BR, 1), lambda i: (i, 0)),
        pl.BlockSpec((1, H), lambda i: (0, 0)),
        pl.BlockSpec((1, H), lambda i: (0, 0)),
        pl.BlockSpec((1, H), lambda i: (0, 0)),
        pl.BlockSpec((H, H // 2), lambda i: (0, 0)),
        pl.BlockSpec((1, H // 2), lambda i: (0, 0)),
        pl.BlockSpec((H // 2, C), lambda i: (0, 0)),
        pl.BlockSpec((1, C), lambda i: (0, 0)),
    ],
    out_specs=pl.BlockSpec((1, C), lambda i: (0, 0)),
    out_shape=jax.ShapeDtypeStruct((1, C), jnp.float32),
    scratch_shapes=[pltpu.VMEM((1, H), jnp.float32)],
)


def kernel(x, edge_index, W0, b0, g0, be0, W1, b1, g1, be1,
           W2, b2, g2, be2, Wc1, bc1, Wc2, bc2):
    edg = edge_index.reshape(2, NW, NCHUNK, CHUNK).transpose(1, 2, 0, 3)
    dst = edge_index[1].reshape(NW, DNCHUNK, DCHUNK)

    degp = _deg(dst)
    d0 = degp[:N].reshape(N, 1)
    d1 = degp[NPAD:NPAD + N].reshape(N, 1)
    zrows = jnp.zeros((ROWS_PER_SUB, D), jnp.float32)

    y0, dinv = _t0(x, W0, d0, d1)
    a0 = _agg(y0, edg, zrows)
    y1 = _tmid(a0, y0, dinv, b0.reshape(1, H), g0.reshape(1, H),
               be0.reshape(1, H), W1)
    a1 = _agg(y1, edg, zrows)
    y2 = _tmid(a1, y1, dinv, b1.reshape(1, H), g1.reshape(1, H),
               be1.reshape(1, H), W2)
    a2 = _agg(y2, edg, zrows)
    out = _t3(a2, y2, dinv, b2.reshape(1, H), g2.reshape(1, H),
              be2.reshape(1, H), Wc1, bc1.reshape(1, H // 2),
              Wc2, bc2.reshape(1, C))
    return out
